# Initial kernel scaffold; baseline (speedup 1.0000x reference)
import functools

import jax
import jax.numpy as jnp
from jax import lax
from jax.experimental import pallas as pl
from jax.experimental.pallas import tpu as pltpu
from jax.experimental.pallas import tpu_sc as plsc

_VOCAB = 128
_D = 32
_ROWS = 4096
_COLS = 200
_B = _ROWS * _COLS
_NC = 2
_NS = 16
_NW = _NC * _NS
_BPW = _B // _NW
_IDXW = 128
_K = 3200
_NSUB = _K // _IDXW
_NCH = _BPW // _K
_IDX_ROWS_PER_W = _BPW // _IDXW


def _emb_body(ids_hbm, table_hbm, out_hbm, idx_v, rows_v, sem):
    wid = lax.axis_index("s") * _NC + lax.axis_index("c")
    idx_base = wid * _IDX_ROWS_PER_W
    out_base = wid * _BPW

    def chunk_body(i, carry):
        pltpu.sync_copy(ids_hbm.at[pl.ds(idx_base + i * _NSUB, _NSUB)], idx_v)

        def fire(j, c):
            pltpu.async_copy(
                table_hbm.at[idx_v.at[j]],
                rows_v.at[pl.ds(j * _IDXW, _IDXW)],
                sem,
            )
            return c

        lax.fori_loop(0, _NSUB, fire, 0)

        def drain(j, c):
            pltpu.make_async_copy(
                table_hbm.at[idx_v.at[j]],
                rows_v.at[pl.ds(j * _IDXW, _IDXW)],
                sem,
            ).wait()
            return c

        lax.fori_loop(0, _NSUB, drain, 0)

        pltpu.sync_copy(rows_v, out_hbm.at[pl.ds(out_base + i * _K, _K)])
        return carry

    lax.fori_loop(0, _NCH, chunk_body, 0)


_emb = functools.partial(
    pl.kernel,
    mesh=plsc.VectorSubcoreMesh(core_axis_name="c", subcore_axis_name="s"),
    out_type=jax.ShapeDtypeStruct((_B, _D), jnp.float32),
    scratch_types=[
        pltpu.VMEM((_NSUB, _IDXW), jnp.int32),
        pltpu.VMEM((_K, _D), jnp.float32),
        pltpu.SemaphoreType.DMA,
    ],
)(_emb_body)


@jax.jit
def kernel(input_ids, embed_weight):
    ids = input_ids.astype(jnp.int32).reshape(_B // _IDXW, _IDXW)
    out = _emb(ids, embed_weight)
    return out.reshape(_ROWS, _COLS, _D)


# speedup vs baseline: 2.9642x; 2.9642x over previous
import functools

import jax
import jax.numpy as jnp
from jax import lax
from jax.experimental import pallas as pl
from jax.experimental.pallas import tpu as pltpu
from jax.experimental.pallas import tpu_sc as plsc

_VOCAB = 128
_D = 32
_ROWS = 4096
_COLS = 200
_B = _ROWS * _COLS
_NC = 2
_NS = 16
_NW = _NC * _NS
_BPW = _B // _NW
_IDXW = 128
_K = 1024
_NSUB = _K // _IDXW
_NCH = _BPW // _K
_IDX_ROWS_PER_W = _BPW // _IDXW


def _emb_body(ids_hbm, table_hbm, out_hbm, idx_v, rows_v, sem):
    wid = lax.axis_index("s") * _NC + lax.axis_index("c")
    idx_base = wid * _IDX_ROWS_PER_W
    out_base = wid * _BPW

    def chunk_body(i, carry):
        pltpu.sync_copy(ids_hbm.at[pl.ds(idx_base + i * _NSUB, _NSUB)], idx_v)

        def fire(j, c):
            pltpu.async_copy(
                table_hbm.at[idx_v.at[j]],
                rows_v.at[pl.ds(j * _IDXW, _IDXW)],
                sem,
            )
            return c

        lax.fori_loop(0, _NSUB, fire, 0)

        def drain(j, c):
            pltpu.make_async_copy(
                table_hbm.at[idx_v.at[j]],
                rows_v.at[pl.ds(j * _IDXW, _IDXW)],
                sem,
            ).wait()
            return c

        lax.fori_loop(0, _NSUB, drain, 0)

        pltpu.sync_copy(rows_v, out_hbm.at[pl.ds(out_base + i * _K, _K)])
        return carry

    lax.fori_loop(0, _NCH, chunk_body, 0)


_emb = functools.partial(
    pl.kernel,
    mesh=plsc.VectorSubcoreMesh(core_axis_name="c", subcore_axis_name="s"),
    out_type=jax.ShapeDtypeStruct((_B, _D), jnp.float32),
    scratch_types=[
        pltpu.VMEM((_NSUB, _IDXW), jnp.int32),
        pltpu.VMEM((_K, _D), jnp.float32),
        pltpu.SemaphoreType.DMA,
    ],
    compiler_params=pltpu.CompilerParams(use_tc_tiling_on_sc=False),
)(_emb_body)


@jax.jit
def kernel(input_ids, embed_weight):
    ids = input_ids.astype(jnp.int32).reshape(_B // _IDXW, _IDXW)
    out = _emb(ids, embed_weight)
    return out.reshape(_ROWS, _COLS, _D)
